# baseline (speedup 1.0000x reference)
import functools

import jax
import jax.numpy as jnp
from jax import lax
from jax.experimental import pallas as pl
from jax.experimental.pallas import tpu as pltpu
from jax.experimental.pallas import tpu_sc as plsc

B = 16384
EMB = 32
LINE = 128
RPL = LINE // EMB


def _make_sc_gather():
    info = plsc.get_sparse_core_info()
    nw = info.num_cores * info.num_subcores
    bpw = B // nw
    mesh = plsc.VectorSubcoreMesh(core_axis_name="c", subcore_axis_name="s")

    @functools.partial(
        pl.kernel,
        out_type=[
            jax.ShapeDtypeStruct((B, LINE), jnp.float32),
            jax.ShapeDtypeStruct((B, LINE), jnp.float32),
        ],
        mesh=mesh,
        scratch_types=[
            pltpu.VMEM((bpw,), jnp.int32),
            pltpu.VMEM((bpw,), jnp.int32),
            pltpu.VMEM((bpw, LINE), jnp.float32),
            pltpu.SemaphoreType.DMA,
        ],
    )
    def sc_gather(ulid_hbm, ilid_hbm, utab_hbm, itab_hbm, uout_hbm, iout_hbm,
                  uidx_v, iidx_v, rows_v, sem):
        wid = lax.axis_index("s") * info.num_cores + lax.axis_index("c")
        base = wid * bpw
        pltpu.sync_copy(ulid_hbm.at[pl.ds(base, bpw)], uidx_v)
        pltpu.sync_copy(ilid_hbm.at[pl.ds(base, bpw)], iidx_v)
        pltpu.async_copy(utab_hbm.at[uidx_v], rows_v, sem).wait()
        pltpu.sync_copy(rows_v, uout_hbm.at[pl.ds(base, bpw)])
        pltpu.async_copy(itab_hbm.at[iidx_v], rows_v, sem).wait()
        pltpu.sync_copy(rows_v, iout_hbm.at[pl.ds(base, bpw)])

    return sc_gather


_sc_gather = _make_sc_gather()


_BLK = 2048


def _select_chunk(g, off):
    acc = jnp.where(off == 0, g[:, 0 * EMB:1 * EMB], 0.0)
    for k in range(1, RPL):
        acc = acc + jnp.where(off == k, g[:, k * EMB:(k + 1) * EMB], 0.0)
    return acc


def _mlp_body(gu_ref, gi_ref, uoff_ref, ioff_ref, w1a_ref, w1b_ref, b1_ref,
              w2_ref, b2_ref, w3_ref, b3_ref, wo_ref, bo_ref, out_ref):
    u = _select_chunk(gu_ref[...], uoff_ref[...])
    i = _select_chunk(gi_ref[...], ioff_ref[...])
    h = u @ w1a_ref[...] + i @ w1b_ref[...] + b1_ref[...]
    h = jnp.maximum(h, 0.0)
    h = jnp.maximum(h @ w2_ref[...] + b2_ref[...], 0.0)
    h = jnp.maximum(h @ w3_ref[...] + b3_ref[...], 0.0)
    out_ref[...] = h @ wo_ref[...] + bo_ref[...]


def _mlp(gu, gi, uoff, ioff, w1a, w1b, b1, w2t, b2, w3t, b3, wot, bo):
    grid = (B // _BLK,)
    full = lambda g: (0, 0)
    return pl.pallas_call(
        _mlp_body,
        grid=grid,
        in_specs=[
            pl.BlockSpec((_BLK, LINE), lambda g: (g, 0)),
            pl.BlockSpec((_BLK, LINE), lambda g: (g, 0)),
            pl.BlockSpec((_BLK, 1), lambda g: (g, 0)),
            pl.BlockSpec((_BLK, 1), lambda g: (g, 0)),
            pl.BlockSpec(w1a.shape, full),
            pl.BlockSpec(w1b.shape, full),
            pl.BlockSpec(b1.shape, full),
            pl.BlockSpec(w2t.shape, full),
            pl.BlockSpec(b2.shape, full),
            pl.BlockSpec(w3t.shape, full),
            pl.BlockSpec(b3.shape, full),
            pl.BlockSpec(wot.shape, full),
            pl.BlockSpec(bo.shape, full),
        ],
        out_specs=pl.BlockSpec((_BLK, 1), lambda g: (g, 0)),
        out_shape=jax.ShapeDtypeStruct((B, 1), jnp.float32),
    )(gu, gi, uoff, ioff, w1a, w1b, b1, w2t, b2, w3t, b3, wot, bo)


@jax.jit
def kernel(user_indices, item_indices, user_table, item_table,
           W1, b1, W2, b2, W3, b3, Wo, bo):
    uidx = user_indices.astype(jnp.int32)
    iidx = item_indices.astype(jnp.int32)
    ulid = uidx // RPL
    ilid = iidx // RPL
    uoff = (uidx % RPL).reshape(B, 1)
    ioff = (iidx % RPL).reshape(B, 1)
    utab = user_table.reshape(-1, LINE)
    itab = item_table.reshape(-1, LINE)
    gu, gi = _sc_gather(ulid, ilid, utab, itab)
    w1a = W1[:, :EMB].T
    w1b = W1[:, EMB:].T
    out = _mlp(gu, gi, uoff, ioff, w1a, w1b, b1.reshape(1, -1),
               W2.T, b2.reshape(1, -1), W3.T, b3.reshape(1, -1),
               Wo.T, bo.reshape(1, -1))
    return out.reshape(B)
